# baseline (speedup 1.0000x reference)
import functools

import jax
import jax.numpy as jnp
from jax import lax
from jax.experimental import pallas as pl
from jax.experimental.pallas import tpu as pltpu
from jax.experimental.pallas import tpu_sc as plsc

N = 4096
H = 1024
I = 2816
E_ROUTED = 7
EPAD = 128
B = 256
NR_ROWS = 2 * N + E_ROUTED * B
NROWS = N + NR_ROWS
NBLK = NROWS // B

BNR = 1024
ASG = 2 * N

DW = 16
DCHUNK = ASG // DW
DFILL = N // DW

GW = 32


def _iota16():
    return lax.broadcasted_iota(jnp.int32, (16,), 0)


def _router_body(x_ref, wr_ref, rb_ref, e1_ref, e2_ref, s1_ref, s2_ref):
    logits = jnp.dot(x_ref[...], wr_ref[...],
                     preferred_element_type=jnp.float32) + rb_ref[...]
    idx = lax.broadcasted_iota(jnp.int32, logits.shape, 1)
    m1 = jnp.max(logits, axis=1, keepdims=True)
    am1 = jnp.min(jnp.where(logits == m1, idx, EPAD), axis=1, keepdims=True)
    mask1 = idx == am1
    l2 = jnp.where(mask1, -1e30, logits)
    m2 = jnp.max(l2, axis=1, keepdims=True)
    am2 = jnp.min(jnp.where(l2 == m2, idx, EPAD), axis=1, keepdims=True)
    e2v = jnp.exp(m2 - m1)
    s1 = 1.0 / (1.0 + e2v)
    s2 = e2v * s1
    shp = logits.shape
    e1_ref[...] = jnp.broadcast_to(am1, shp)
    e2_ref[...] = jnp.broadcast_to(am2, shp)
    s1_ref[...] = jnp.broadcast_to(s1, shp)
    s2_ref[...] = jnp.broadcast_to(s2, shp)


def _router(xbf, Wr_pad, rb_pad):
    f = jax.ShapeDtypeStruct((N, EPAD), jnp.float32)
    i = jax.ShapeDtypeStruct((N, EPAD), jnp.int32)
    return pl.pallas_call(
        _router_body,
        grid=(N // BNR,),
        in_specs=[
            pl.BlockSpec((BNR, H), lambda t: (t, 0)),
            pl.BlockSpec((H, EPAD), lambda t: (0, 0)),
            pl.BlockSpec((1, EPAD), lambda t: (0, 0)),
        ],
        out_specs=[pl.BlockSpec((BNR, EPAD), lambda t: (t, 0))] * 4,
        out_shape=[i, i, f, f],
        compiler_params=pltpu.CompilerParams(
            dimension_semantics=("parallel",)),
    )(xbf, Wr_pad, rb_pad)


def _dispatch_body(eids_hbm, rt_hbm, pos_hbm, bexp_hbm,
                   ev, fillbuf, posf, tokf, posb, tokb, bexpb, sem):
    wid = lax.axis_index("s")
    iota = _iota16()

    pltpu.sync_copy(eids_hbm, ev)

    c = jnp.zeros((16,), jnp.int32)
    pref = jnp.zeros((16,), jnp.int32)
    for ch in range(DW):
        pref = jnp.where(ch == wid, c, pref)

        def inner(v, accs):
            lanev = ev[pl.ds(ch * DCHUNK + v * 16, 16)]
            return tuple(accs[e] + jnp.where(lanev == e, 1, 0)
                         for e in range(E_ROUTED))
        accs = lax.fori_loop(0, DCHUNK // 16, inner,
                             tuple(jnp.zeros((16,), jnp.int32)
                                   for _ in range(E_ROUTED)))
        for e in range(E_ROUTED):
            c = c + jnp.where(iota == e, jnp.sum(accs[e]), 0)
    tot = c

    padded = ((tot + (B - 1)) >> 8) << 8
    incl = plsc.cumsum(padded)
    base = N + incl - padded
    startv = base + pref
    cum_end = N + incl

    def fill_body(v, carry):
        fillbuf[pl.ds(v * 16, 16)] = wid * DFILL + v * 16 + iota
        return carry
    lax.fori_loop(0, DFILL // 16, fill_body, 0)
    pltpu.sync_copy(fillbuf, rt_hbm.at[pl.ds(wid * DFILL, DFILL)])

    def dest_body(v, sv):
        lanev = ev[pl.ds(wid * DCHUNK + v * 16, 16)]
        j = wid * DCHUNK + v * 16 + iota
        tok = jnp.bitwise_and(j, N - 1)
        dest = jnp.zeros((16,), jnp.int32)
        for e in range(E_ROUTED):
            m = lanev == e
            mi = jnp.where(m, 1, 0).astype(jnp.int32)
            incl_m = plsc.cumsum(mi)
            cnt_e = jnp.sum(mi)
            se = jnp.sum(jnp.where(iota == e, sv, 0))
            dest = jnp.where(m, se + incl_m - 1, dest)
            sv = sv + jnp.where(iota == e, cnt_e, 0)
        posf[pl.ds(v * 16, 16)] = dest
        tokf[pl.ds(v * 16, 16)] = tok
        return sv
    lax.fori_loop(0, DCHUNK // 16, dest_body, startv)

    for v in range(DCHUNK // 16):
        cc, rr = divmod(v, 8)
        posb[cc, pl.ds(rr * 16, 16)] = posf[pl.ds(v * 16, 16)]
        tokb[cc, pl.ds(rr * 16, 16)] = tokf[pl.ds(v * 16, 16)]

    pltpu.sync_copy(posb, pos_hbm.at[pl.ds((DCHUNK // 128) * wid, DCHUNK // 128)])
    handles = [pltpu.async_copy(tokb.at[cc], rt_hbm.at[posb.at[cc]], sem)
               for cc in range(DCHUNK // 128)]
    for h in handles:
        h.wait()

    @pl.when(wid == 0)
    def _():
        for q in range(64 // 16):
            bv = q * 16 + iota
            rs = bv * B
            acc = jnp.zeros((16,), jnp.int32)
            for e in range(E_ROUTED):
                ce = jnp.sum(jnp.where(iota == e, cum_end, 0))
                acc = acc + jnp.where(rs >= ce, 1, 0)
            bexpb[pl.ds(q * 16, 16)] = jnp.where(
                bv < (N // B), 0, 1 + jnp.minimum(acc, E_ROUTED - 1))
        pltpu.sync_copy(bexpb, bexp_hbm)


def _dispatch(eids):
    mesh = plsc.VectorSubcoreMesh(core_axis_name="c", subcore_axis_name="s",
                                  num_cores=1)
    return pl.kernel(
        _dispatch_body,
        mesh=mesh,
        compiler_params=pltpu.CompilerParams(needs_layout_passes=False),
        out_type=(
            jax.ShapeDtypeStruct((NROWS,), jnp.int32),
            jax.ShapeDtypeStruct((ASG // 128, 128), jnp.int32),
            jax.ShapeDtypeStruct((64,), jnp.int32),
        ),
        scratch_types=[
            pltpu.VMEM((ASG,), jnp.int32),
            pltpu.VMEM((DFILL,), jnp.int32),
            pltpu.VMEM((DCHUNK,), jnp.int32),
            pltpu.VMEM((DCHUNK,), jnp.int32),
            pltpu.VMEM((DCHUNK // 128, 128), jnp.int32),
            pltpu.VMEM((DCHUNK // 128, 128), jnp.int32),
            pltpu.VMEM((64,), jnp.int32),
            pltpu.SemaphoreType.DMA,
        ],
    )(eids)


def _bf16_to_i32(a2d):
    n = a2d.shape[0]
    return lax.bitcast_convert_type(
        a2d.reshape(n, H // 2, 2), jnp.int32).reshape(n, 4, 128)


def _i32_to_bf16(a3d):
    n = a3d.shape[0]
    return lax.bitcast_convert_type(
        a3d.reshape(n, H // 2), jnp.bfloat16).reshape(n, H)


def _gather_rows(table3, idx, rows, chunk):
    m = rows // GW
    nch = m // chunk
    mesh = plsc.VectorSubcoreMesh(core_axis_name="c", subcore_axis_name="s",
                                  num_cores=2)

    rmax = table3.shape[0] - 1

    def body(tab_hbm, idx_hbm, out_hbm, idxv, buf, sem):
        wid = lax.axis_index("s") * 2 + lax.axis_index("c")
        base = wid * m
        pltpu.sync_copy(idx_hbm.at[pl.ds(base, m)], idxv)
        starts = sorted({min(v * 16, m - 16) for v in range((m + 15) // 16)})
        for st in starts:
            ival = idxv[pl.ds(st, 16)]
            idxv[pl.ds(st, 16)] = jnp.minimum(jnp.maximum(ival, 0), rmax)
        for ci in range(nch):
            pltpu.async_copy(
                tab_hbm.at[idxv.at[pl.ds(ci * chunk, chunk)]], buf, sem
            ).wait()
            pltpu.sync_copy(buf, out_hbm.at[pl.ds(base + ci * chunk, chunk)])

    return pl.kernel(
        body,
        mesh=mesh,
        compiler_params=pltpu.CompilerParams(needs_layout_passes=False),
        out_type=jax.ShapeDtypeStruct((rows, 4, 128), jnp.int32),
        scratch_types=[
            pltpu.VMEM((m,), jnp.int32),
            pltpu.VMEM((chunk, 4, 128), jnp.int32),
            pltpu.SemaphoreType.DMA,
        ],
    )(table3, idx)


def _gmm_body(bexp_ref, xd_ref, wg_ref, wu_ref, wd_ref, eo_ref):
    xb = xd_ref[...]
    g = jnp.dot(xb, wg_ref[0], preferred_element_type=jnp.float32)
    u = jnp.dot(xb, wu_ref[0], preferred_element_type=jnp.float32)
    h = (g * jax.nn.sigmoid(g) * u).astype(jnp.bfloat16)
    eo_ref[...] = jnp.dot(
        h, wd_ref[0], preferred_element_type=jnp.float32).astype(jnp.bfloat16)


def _gmm(bexp, xd, WgA, WuA, WdA):
    grid_spec = pltpu.PrefetchScalarGridSpec(
        num_scalar_prefetch=1,
        grid=(NBLK,),
        in_specs=[
            pl.BlockSpec((B, H), lambda b, be: (b, 0)),
            pl.BlockSpec((1, H, I), lambda b, be: (be[b], 0, 0)),
            pl.BlockSpec((1, H, I), lambda b, be: (be[b], 0, 0)),
            pl.BlockSpec((1, I, H), lambda b, be: (be[b], 0, 0)),
        ],
        out_specs=pl.BlockSpec((B, H), lambda b, be: (b, 0)),
    )
    return pl.pallas_call(
        _gmm_body,
        grid_spec=grid_spec,
        out_shape=jax.ShapeDtypeStruct((NROWS, H), jnp.bfloat16),
        compiler_params=pltpu.CompilerParams(
            dimension_semantics=("arbitrary",)),
    )(bexp, xd, WgA, WuA, WdA)


def _combine_body(eo_ref, g1_ref, g2_ref, s1_ref, s2_ref, out_ref):
    w1 = s1_ref[:, 0:1]
    w2 = s2_ref[:, 0:1]
    out_ref[...] = (eo_ref[...].astype(jnp.float32)
                    + w1 * g1_ref[...].astype(jnp.float32)
                    + w2 * g2_ref[...].astype(jnp.float32))


def _combine(eo, G, S1, S2):
    blk = 512
    return pl.pallas_call(
        _combine_body,
        grid=(N // blk,),
        in_specs=[
            pl.BlockSpec((blk, H), lambda t: (t, 0)),
            pl.BlockSpec((blk, H), lambda t: (t, 0)),
            pl.BlockSpec((blk, H), lambda t: (t + N // blk, 0)),
            pl.BlockSpec((blk, EPAD), lambda t: (t, 0)),
            pl.BlockSpec((blk, EPAD), lambda t: (t, 0)),
        ],
        out_specs=pl.BlockSpec((blk, H), lambda t: (t, 0)),
        out_shape=jax.ShapeDtypeStruct((N, H), jnp.float32),
        compiler_params=pltpu.CompilerParams(
            dimension_semantics=("parallel",)),
    )(eo, G, G, S1, S2)


def kernel(x, Wg_s, Wu_s, Wd_s, Wg, Wu, Wd, Wr, rbias):
    bf = jnp.bfloat16
    xbf = x.astype(bf)
    WgA = jnp.concatenate([Wg_s[None], Wg], axis=0).astype(bf)
    WuA = jnp.concatenate([Wu_s[None], Wu], axis=0).astype(bf)
    WdA = jnp.concatenate([Wd_s[None], Wd], axis=0).astype(bf)
    Wr_pad = jnp.zeros((H, EPAD), bf).at[:, :E_ROUTED].set(Wr.astype(bf))
    rb_pad = jnp.full((1, EPAD), -1e30, jnp.float32).at[0, :E_ROUTED].set(rbias)

    E1, E2, S1, S2 = _router(xbf, Wr_pad, rb_pad)
    eids = jnp.concatenate([E1[:, 0], E2[:, 0]])
    rt, pos2d, bexp = _dispatch(eids)

    x3 = _bf16_to_i32(xbf)
    xd = _i32_to_bf16(_gather_rows(x3, rt, NROWS, 88))
    eo = _gmm(bexp, xd, WgA, WuA, WdA)
    G = _i32_to_bf16(_gather_rows(_bf16_to_i32(eo), pos2d.reshape(-1),
                                  ASG, 128))
    return _combine(eo, G, S1, S2)


# speedup vs baseline: 1.9456x; 1.9456x over previous
import functools

import jax
import jax.numpy as jnp
from jax import lax
from jax.experimental import pallas as pl
from jax.experimental.pallas import tpu as pltpu
from jax.experimental.pallas import tpu_sc as plsc

N = 4096
H = 1024
I = 2816
E_ROUTED = 7
EPAD = 128
B = 256
R_ROWS = 2 * N + E_ROUTED * B
NBLK = R_ROWS // B

BNR = 1024
ASG = 2 * N

DW = 16
DCHUNK = ASG // DW

GW = 32

_VMEM_LIMIT = 63 * 1024 * 1024


def _iota16():
    return lax.broadcasted_iota(jnp.int32, (16,), 0)


def _router_body(x_ref, wr_ref, rb_ref, e1_ref, e2_ref, s1_ref, s2_ref):
    logits = jnp.dot(x_ref[...].astype(jnp.bfloat16),
                     wr_ref[...].astype(jnp.bfloat16),
                     preferred_element_type=jnp.float32) + rb_ref[...]
    idx = lax.broadcasted_iota(jnp.int32, logits.shape, 1)
    m1 = jnp.max(logits, axis=1, keepdims=True)
    am1 = jnp.min(jnp.where(logits == m1, idx, EPAD), axis=1, keepdims=True)
    mask1 = idx == am1
    l2 = jnp.where(mask1, -1e30, logits)
    m2 = jnp.max(l2, axis=1, keepdims=True)
    am2 = jnp.min(jnp.where(l2 == m2, idx, EPAD), axis=1, keepdims=True)
    e2v = jnp.exp(m2 - m1)
    s1 = 1.0 / (1.0 + e2v)
    s2 = e2v * s1
    shp = logits.shape
    e1_ref[...] = jnp.broadcast_to(am1, shp)
    e2_ref[...] = jnp.broadcast_to(am2, shp)
    s1_ref[...] = jnp.broadcast_to(s1, shp)
    s2_ref[...] = jnp.broadcast_to(s2, shp)


def _router(x, Wr_pad, rb_pad):
    f = jax.ShapeDtypeStruct((N, EPAD), jnp.float32)
    i = jax.ShapeDtypeStruct((N, EPAD), jnp.int32)
    return pl.pallas_call(
        _router_body,
        grid=(N // BNR,),
        in_specs=[
            pl.BlockSpec((BNR, H), lambda t: (t, 0)),
            pl.BlockSpec((H, EPAD), lambda t: (0, 0)),
            pl.BlockSpec((1, EPAD), lambda t: (0, 0)),
        ],
        out_specs=[pl.BlockSpec((BNR, EPAD), lambda t: (t, 0))] * 4,
        out_shape=[i, i, f, f],
        compiler_params=pltpu.CompilerParams(
            dimension_semantics=("parallel",)),
    )(x, Wr_pad, rb_pad)


def _dispatch_body(eids_hbm, rt_hbm, pos_hbm, bexp_hbm,
                   ev, posf, tokf, posb, tokb, bexpb, sem):
    wid = lax.axis_index("s")
    iota = _iota16()

    pltpu.sync_copy(eids_hbm, ev)

    c = jnp.zeros((16,), jnp.int32)
    pref = jnp.zeros((16,), jnp.int32)
    for ch in range(DW):
        pref = jnp.where(ch == wid, c, pref)

        def inner(v, accs):
            lanev = ev[pl.ds(ch * DCHUNK + v * 16, 16)]
            return tuple(accs[e] + jnp.where(lanev == e, 1, 0)
                         for e in range(E_ROUTED))
        accs = lax.fori_loop(0, DCHUNK // 16, inner,
                             tuple(jnp.zeros((16,), jnp.int32)
                                   for _ in range(E_ROUTED)))
        for e in range(E_ROUTED):
            c = c + jnp.where(iota == e, jnp.sum(accs[e]), 0)
    tot = c

    padded = ((tot + (B - 1)) >> 8) << 8
    incl = plsc.cumsum(padded)
    base = incl - padded
    startv = base + pref
    cum_end = incl

    def dest_body(v, sv):
        lanev = ev[pl.ds(wid * DCHUNK + v * 16, 16)]
        j = wid * DCHUNK + v * 16 + iota
        tok = jnp.bitwise_and(j, N - 1)
        dest = jnp.zeros((16,), jnp.int32)
        for e in range(E_ROUTED):
            m = lanev == e
            mi = jnp.where(m, 1, 0).astype(jnp.int32)
            incl_m = plsc.cumsum(mi)
            cnt_e = jnp.sum(mi)
            se = jnp.sum(jnp.where(iota == e, sv, 0))
            dest = jnp.where(m, se + incl_m - 1, dest)
            sv = sv + jnp.where(iota == e, cnt_e, 0)
        posf[pl.ds(v * 16, 16)] = dest
        tokf[pl.ds(v * 16, 16)] = tok
        return sv
    lax.fori_loop(0, DCHUNK // 16, dest_body, startv)

    for v in range(DCHUNK // 16):
        cc, rr = divmod(v, 8)
        posb[cc, pl.ds(rr * 16, 16)] = posf[pl.ds(v * 16, 16)]
        tokb[cc, pl.ds(rr * 16, 16)] = tokf[pl.ds(v * 16, 16)]

    pltpu.sync_copy(posb, pos_hbm.at[pl.ds((DCHUNK // 128) * wid, DCHUNK // 128)])
    handles = [pltpu.async_copy(tokb.at[cc], rt_hbm.at[posb.at[cc]], sem)
               for cc in range(DCHUNK // 128)]
    for h in handles:
        h.wait()

    @pl.when(wid == 0)
    def _():
        for q in range(64 // 16):
            bv = q * 16 + iota
            rs = bv * B
            acc = jnp.zeros((16,), jnp.int32)
            for e in range(E_ROUTED):
                ce = jnp.sum(jnp.where(iota == e, cum_end, 0))
                acc = acc + jnp.where(rs >= ce, 1, 0)
            bexpb[pl.ds(q * 16, 16)] = jnp.minimum(acc, E_ROUTED - 1)
        pltpu.sync_copy(bexpb, bexp_hbm)


def _dispatch(eids):
    mesh = plsc.VectorSubcoreMesh(core_axis_name="c", subcore_axis_name="s",
                                  num_cores=1)
    return pl.kernel(
        _dispatch_body,
        mesh=mesh,
        compiler_params=pltpu.CompilerParams(needs_layout_passes=False),
        out_type=(
            jax.ShapeDtypeStruct((R_ROWS,), jnp.int32),
            jax.ShapeDtypeStruct((ASG // 128, 128), jnp.int32),
            jax.ShapeDtypeStruct((64,), jnp.int32),
        ),
        scratch_types=[
            pltpu.VMEM((ASG,), jnp.int32),
            pltpu.VMEM((DCHUNK,), jnp.int32),
            pltpu.VMEM((DCHUNK,), jnp.int32),
            pltpu.VMEM((DCHUNK // 128, 128), jnp.int32),
            pltpu.VMEM((DCHUNK // 128, 128), jnp.int32),
            pltpu.VMEM((64,), jnp.int32),
            pltpu.SemaphoreType.DMA,
        ],
    )(eids)


def _gather_rows(table3, idx, rows, chunk):
    m = rows // GW
    nch = m // chunk
    rmax = table3.shape[0] - 1
    mesh = plsc.VectorSubcoreMesh(core_axis_name="c", subcore_axis_name="s",
                                  num_cores=2)

    def body(tab_hbm, idx_hbm, out_hbm, idxv, buf0, buf1, sem0, sem1):
        wid = lax.axis_index("s") * 2 + lax.axis_index("c")
        base = wid * m
        bufs, sems = (buf0, buf1), (sem0, sem1)
        pltpu.sync_copy(idx_hbm.at[pl.ds(base, m)], idxv)
        starts = sorted({min(v * 16, m - 16) for v in range((m + 15) // 16)})
        for st in starts:
            ival = idxv[pl.ds(st, 16)]
            idxv[pl.ds(st, 16)] = jnp.minimum(jnp.maximum(ival, 0), rmax)
        h = pltpu.async_copy(
            tab_hbm.at[idxv.at[pl.ds(0, chunk)]], bufs[0], sems[0])
        for ci in range(1, nch):
            h2 = pltpu.async_copy(
                tab_hbm.at[idxv.at[pl.ds(ci * chunk, chunk)]],
                bufs[ci % 2], sems[ci % 2])
            h.wait()
            pltpu.sync_copy(bufs[(ci - 1) % 2],
                            out_hbm.at[pl.ds(base + (ci - 1) * chunk, chunk)])
            h = h2
        h.wait()
        pltpu.sync_copy(bufs[(nch - 1) % 2],
                        out_hbm.at[pl.ds(base + (nch - 1) * chunk, chunk)])

    return pl.kernel(
        body,
        mesh=mesh,
        compiler_params=pltpu.CompilerParams(needs_layout_passes=False),
        out_type=jax.ShapeDtypeStruct((rows, 8, 128), jnp.float32),
        scratch_types=[
            pltpu.VMEM((m,), jnp.int32),
            pltpu.VMEM((chunk, 8, 128), jnp.float32),
            pltpu.VMEM((chunk, 8, 128), jnp.float32),
            pltpu.SemaphoreType.DMA,
            pltpu.SemaphoreType.DMA,
        ],
    )(table3, idx)


def _gmm_body(bexp_ref, xd_ref, wg_ref, wu_ref, wd_ref, eo_ref):
    bf = jnp.bfloat16
    xb = xd_ref[...].astype(bf)
    g = jnp.dot(xb, wg_ref[0], preferred_element_type=jnp.float32)
    u = jnp.dot(xb, wu_ref[0], preferred_element_type=jnp.float32)
    h = (g * jax.nn.sigmoid(g) * u).astype(bf)
    eo_ref[...] = jnp.dot(h, wd_ref[0], preferred_element_type=jnp.float32)


def _gmm(bexp, xd, Wg, Wu, Wd):
    grid_spec = pltpu.PrefetchScalarGridSpec(
        num_scalar_prefetch=1,
        grid=(NBLK,),
        in_specs=[
            pl.BlockSpec((B, H), lambda b, be: (b, 0)),
            pl.BlockSpec((1, H, I), lambda b, be: (be[b], 0, 0)),
            pl.BlockSpec((1, H, I), lambda b, be: (be[b], 0, 0)),
            pl.BlockSpec((1, I, H), lambda b, be: (be[b], 0, 0)),
        ],
        out_specs=pl.BlockSpec((B, H), lambda b, be: (b, 0)),
    )
    return pl.pallas_call(
        _gmm_body,
        grid_spec=grid_spec,
        out_shape=jax.ShapeDtypeStruct((R_ROWS, H), jnp.float32),
        compiler_params=pltpu.CompilerParams(
            dimension_semantics=("arbitrary",),
            vmem_limit_bytes=_VMEM_LIMIT),
    )(bexp, xd, Wg, Wu, Wd)


def _shared_body(x_ref, wg_ref, wu_ref, wd_ref, out_ref):
    bf = jnp.bfloat16
    xb = x_ref[...].astype(bf)
    g = jnp.dot(xb, wg_ref[...], preferred_element_type=jnp.float32)
    u = jnp.dot(xb, wu_ref[...], preferred_element_type=jnp.float32)
    h = (g * jax.nn.sigmoid(g) * u).astype(bf)
    out_ref[...] = jnp.dot(h, wd_ref[...], preferred_element_type=jnp.float32)


def _shared(x, Wg_s, Wu_s, Wd_s):
    blk = 256
    return pl.pallas_call(
        _shared_body,
        grid=(N // blk,),
        in_specs=[
            pl.BlockSpec((blk, H), lambda t: (t, 0)),
            pl.BlockSpec((H, I), lambda t: (0, 0)),
            pl.BlockSpec((H, I), lambda t: (0, 0)),
            pl.BlockSpec((I, H), lambda t: (0, 0)),
        ],
        out_specs=pl.BlockSpec((blk, H), lambda t: (t, 0)),
        out_shape=jax.ShapeDtypeStruct((N, H), jnp.float32),
        compiler_params=pltpu.CompilerParams(
            dimension_semantics=("arbitrary",),
            vmem_limit_bytes=_VMEM_LIMIT),
    )(x, Wg_s, Wu_s, Wd_s)


def _combine_body(sh_ref, g1_ref, g2_ref, s1_ref, s2_ref, out_ref):
    w1 = s1_ref[:, 0:1]
    w2 = s2_ref[:, 0:1]
    out_ref[...] = sh_ref[...] + w1 * g1_ref[...] + w2 * g2_ref[...]


def _combine(sh, G, S1, S2):
    blk = 512
    return pl.pallas_call(
        _combine_body,
        grid=(N // blk,),
        in_specs=[
            pl.BlockSpec((blk, H), lambda t: (t, 0)),
            pl.BlockSpec((blk, H), lambda t: (t, 0)),
            pl.BlockSpec((blk, H), lambda t: (t + N // blk, 0)),
            pl.BlockSpec((blk, EPAD), lambda t: (t, 0)),
            pl.BlockSpec((blk, EPAD), lambda t: (t, 0)),
        ],
        out_specs=pl.BlockSpec((blk, H), lambda t: (t, 0)),
        out_shape=jax.ShapeDtypeStruct((N, H), jnp.float32),
        compiler_params=pltpu.CompilerParams(
            dimension_semantics=("parallel",)),
    )(sh, G, G, S1, S2)


def kernel(x, Wg_s, Wu_s, Wd_s, Wg, Wu, Wd, Wr, rbias):
    Wr_pad = jnp.zeros((H, EPAD), jnp.float32).at[:, :E_ROUTED].set(Wr)
    rb_pad = jnp.full((1, EPAD), -1e30, jnp.float32).at[0, :E_ROUTED].set(rbias)

    bf = jnp.bfloat16
    E1, E2, S1, S2 = _router(x, Wr_pad, rb_pad)
    eids = jnp.concatenate([E1[:, 0], E2[:, 0]])
    rt, pos2d, bexp = _dispatch(eids)

    x3 = x.reshape(N, 8, 128)
    xd = _gather_rows(x3, rt, R_ROWS, 24).reshape(R_ROWS, H)
    eo = _gmm(bexp, xd, Wg.astype(bf), Wu.astype(bf), Wd.astype(bf))
    sh = _shared(x, Wg_s.astype(bf), Wu_s.astype(bf), Wd_s.astype(bf))
    G = _gather_rows(eo.reshape(R_ROWS, 8, 128), pos2d.reshape(-1),
                     ASG, 32).reshape(ASG, H)
    return _combine(sh, G, S1, S2)


# speedup vs baseline: 2.0180x; 1.0372x over previous
import functools

import jax
import jax.numpy as jnp
from jax import lax
from jax.experimental import pallas as pl
from jax.experimental.pallas import tpu as pltpu
from jax.experimental.pallas import tpu_sc as plsc

N = 4096
H = 1024
I = 2816
E_ROUTED = 7
EPAD = 128
B = 256
R_ROWS = 2 * N + E_ROUTED * B
NBLK = R_ROWS // B

BNR = 1024
ASG = 2 * N

DW = 16
DCHUNK = ASG // DW

GW = 32

_VMEM_LIMIT = 63 * 1024 * 1024


def _iota16():
    return lax.broadcasted_iota(jnp.int32, (16,), 0)


def _router_body(x_ref, wr_ref, rb_ref, e1_ref, e2_ref, s1_ref, s2_ref):
    logits = jnp.dot(x_ref[...].astype(jnp.bfloat16),
                     wr_ref[...].astype(jnp.bfloat16),
                     preferred_element_type=jnp.float32) + rb_ref[...]
    idx = lax.broadcasted_iota(jnp.int32, logits.shape, 1)
    m1 = jnp.max(logits, axis=1, keepdims=True)
    am1 = jnp.min(jnp.where(logits == m1, idx, EPAD), axis=1, keepdims=True)
    mask1 = idx == am1
    l2 = jnp.where(mask1, -1e30, logits)
    m2 = jnp.max(l2, axis=1, keepdims=True)
    am2 = jnp.min(jnp.where(l2 == m2, idx, EPAD), axis=1, keepdims=True)
    e2v = jnp.exp(m2 - m1)
    s1 = 1.0 / (1.0 + e2v)
    s2 = e2v * s1
    shp = logits.shape
    e1_ref[...] = jnp.broadcast_to(am1, shp)
    e2_ref[...] = jnp.broadcast_to(am2, shp)
    s1_ref[...] = jnp.broadcast_to(s1, shp)
    s2_ref[...] = jnp.broadcast_to(s2, shp)


def _router(x, Wr_pad, rb_pad):
    f = jax.ShapeDtypeStruct((N, EPAD), jnp.float32)
    i = jax.ShapeDtypeStruct((N, EPAD), jnp.int32)
    return pl.pallas_call(
        _router_body,
        grid=(N // BNR,),
        in_specs=[
            pl.BlockSpec((BNR, H), lambda t: (t, 0)),
            pl.BlockSpec((H, EPAD), lambda t: (0, 0)),
            pl.BlockSpec((1, EPAD), lambda t: (0, 0)),
        ],
        out_specs=[pl.BlockSpec((BNR, EPAD), lambda t: (t, 0))] * 4,
        out_shape=[i, i, f, f],
        compiler_params=pltpu.CompilerParams(
            dimension_semantics=("parallel",)),
    )(x, Wr_pad, rb_pad)


def _dispatch_body(eids_hbm, rt_hbm, pos_hbm, bexp_hbm,
                   ev, posf, tokf, posb, tokb, bexpb, sem):
    wid = lax.axis_index("s")
    iota = _iota16()

    pltpu.sync_copy(eids_hbm, ev)

    c = jnp.zeros((16,), jnp.int32)
    pref = jnp.zeros((16,), jnp.int32)
    for ch in range(DW):
        pref = jnp.where(ch == wid, c, pref)

        def inner(v, accs):
            lanev = ev[pl.ds(ch * DCHUNK + v * 16, 16)]
            return tuple(accs[e] + jnp.where(lanev == e, 1, 0)
                         for e in range(E_ROUTED))
        accs = lax.fori_loop(0, DCHUNK // 16, inner,
                             tuple(jnp.zeros((16,), jnp.int32)
                                   for _ in range(E_ROUTED)))
        for e in range(E_ROUTED):
            c = c + jnp.where(iota == e, jnp.sum(accs[e]), 0)
    tot = c

    padded = ((tot + (B - 1)) >> 8) << 8
    incl = plsc.cumsum(padded)
    base = incl - padded
    startv = base + pref
    cum_end = incl

    def dest_body(v, sv):
        lanev = ev[pl.ds(wid * DCHUNK + v * 16, 16)]
        j = wid * DCHUNK + v * 16 + iota
        tok = jnp.bitwise_and(j, N - 1)
        dest = jnp.zeros((16,), jnp.int32)
        for e in range(E_ROUTED):
            m = lanev == e
            mi = jnp.where(m, 1, 0).astype(jnp.int32)
            incl_m = plsc.cumsum(mi)
            cnt_e = jnp.sum(mi)
            se = jnp.sum(jnp.where(iota == e, sv, 0))
            dest = jnp.where(m, se + incl_m - 1, dest)
            sv = sv + jnp.where(iota == e, cnt_e, 0)
        posf[pl.ds(v * 16, 16)] = dest
        tokf[pl.ds(v * 16, 16)] = tok
        return sv
    lax.fori_loop(0, DCHUNK // 16, dest_body, startv)

    for v in range(DCHUNK // 16):
        cc, rr = divmod(v, 8)
        posb[cc, pl.ds(rr * 16, 16)] = posf[pl.ds(v * 16, 16)]
        tokb[cc, pl.ds(rr * 16, 16)] = tokf[pl.ds(v * 16, 16)]

    pltpu.sync_copy(posb, pos_hbm.at[pl.ds((DCHUNK // 128) * wid, DCHUNK // 128)])
    handles = [pltpu.async_copy(tokb.at[cc], rt_hbm.at[posb.at[cc]], sem)
               for cc in range(DCHUNK // 128)]
    for h in handles:
        h.wait()

    @pl.when(wid == 0)
    def _():
        for q in range(64 // 16):
            bv = q * 16 + iota
            rs = bv * B
            acc = jnp.zeros((16,), jnp.int32)
            for e in range(E_ROUTED):
                ce = jnp.sum(jnp.where(iota == e, cum_end, 0))
                acc = acc + jnp.where(rs >= ce, 1, 0)
            bexpb[pl.ds(q * 16, 16)] = jnp.minimum(acc, E_ROUTED - 1)
        pltpu.sync_copy(bexpb, bexp_hbm)


def _dispatch(eids):
    mesh = plsc.VectorSubcoreMesh(core_axis_name="c", subcore_axis_name="s",
                                  num_cores=1)
    return pl.kernel(
        _dispatch_body,
        mesh=mesh,
        compiler_params=pltpu.CompilerParams(needs_layout_passes=False),
        out_type=(
            jax.ShapeDtypeStruct((R_ROWS,), jnp.int32),
            jax.ShapeDtypeStruct((ASG // 128, 128), jnp.int32),
            jax.ShapeDtypeStruct((64,), jnp.int32),
        ),
        scratch_types=[
            pltpu.VMEM((ASG,), jnp.int32),
            pltpu.VMEM((DCHUNK,), jnp.int32),
            pltpu.VMEM((DCHUNK,), jnp.int32),
            pltpu.VMEM((DCHUNK // 128, 128), jnp.int32),
            pltpu.VMEM((DCHUNK // 128, 128), jnp.int32),
            pltpu.VMEM((64,), jnp.int32),
            pltpu.SemaphoreType.DMA,
        ],
    )(eids)


def _gather_rows(table3, idx, rows, chunk=48):
    m = rows // GW
    sizes = [chunk] * (m // chunk) + ([m % chunk] if m % chunk else [])
    offs = [sum(sizes[:i]) for i in range(len(sizes))]
    nch = len(sizes)
    rmax = table3.shape[0] - 1
    mesh = plsc.VectorSubcoreMesh(core_axis_name="c", subcore_axis_name="s",
                                  num_cores=2)

    def body(tab_hbm, idx_hbm, out_hbm, idxv, buf0, buf1, gs0, gs1, os0, os1):
        wid = lax.axis_index("s") * 2 + lax.axis_index("c")
        base = wid * m
        bufs, gsems, osems = (buf0, buf1), (gs0, gs1), (os0, os1)
        pltpu.sync_copy(idx_hbm.at[pl.ds(base, m)], idxv)
        starts = sorted({min(v * 16, m - 16) for v in range((m + 15) // 16)})
        for st in starts:
            ival = idxv[pl.ds(st, 16)]
            idxv[pl.ds(st, 16)] = jnp.minimum(jnp.maximum(ival, 0), rmax)

        gh = [None, None]
        oh = [None, None]
        for ci in range(nch):
            s = ci % 2
            if oh[s] is not None:
                oh[s].wait()
                oh[s] = None
            gh[s] = pltpu.async_copy(
                tab_hbm.at[idxv.at[pl.ds(offs[ci], sizes[ci])]],
                bufs[s].at[pl.ds(0, sizes[ci])], gsems[s])
            if ci >= 1:
                p = (ci - 1) % 2
                gh[p].wait()
                oh[p] = pltpu.async_copy(
                    bufs[p].at[pl.ds(0, sizes[ci - 1])],
                    out_hbm.at[pl.ds(base + offs[ci - 1], sizes[ci - 1])],
                    osems[p])
        last = (nch - 1) % 2
        gh[last].wait()
        oh[last] = pltpu.async_copy(
            bufs[last].at[pl.ds(0, sizes[nch - 1])],
            out_hbm.at[pl.ds(base + offs[nch - 1], sizes[nch - 1])],
            osems[last])
        for s in range(2):
            if oh[s] is not None:
                oh[s].wait()

    return pl.kernel(
        body,
        mesh=mesh,
        compiler_params=pltpu.CompilerParams(needs_layout_passes=False),
        out_type=jax.ShapeDtypeStruct((rows, 8, 128), jnp.float32),
        scratch_types=[
            pltpu.VMEM((m,), jnp.int32),
            pltpu.VMEM((chunk, 8, 128), jnp.float32),
            pltpu.VMEM((chunk, 8, 128), jnp.float32),
            pltpu.SemaphoreType.DMA,
            pltpu.SemaphoreType.DMA,
            pltpu.SemaphoreType.DMA,
            pltpu.SemaphoreType.DMA,
        ],
    )(table3, idx)


BI = I // 2


def _gmm_half_body(first, bexp_ref, xd_ref, wg_ref, wu_ref, wd_ref, eo_ref,
                   *maybe_prev):
    bf = jnp.bfloat16
    xb = xd_ref[...].astype(bf)
    g = jnp.dot(xb, wg_ref[0].astype(bf), preferred_element_type=jnp.float32)
    u = jnp.dot(xb, wu_ref[0].astype(bf), preferred_element_type=jnp.float32)
    h = (g * jax.nn.sigmoid(g) * u).astype(bf)
    part = jnp.dot(h, wd_ref[0].astype(bf), preferred_element_type=jnp.float32)
    if first:
        eo_ref[...] = part
    else:
        eo_ref[...] = part + maybe_prev[0][...]


def _gmm_half(ib, bexp, xd, Wg, Wu, Wd, prev=None):
    first = prev is None
    in_specs = [
        pl.BlockSpec((B, H), lambda b, be: (b, 0)),
        pl.BlockSpec((1, H, BI), lambda b, be: (be[b], 0, ib)),
        pl.BlockSpec((1, H, BI), lambda b, be: (be[b], 0, ib)),
        pl.BlockSpec((1, BI, H), lambda b, be: (be[b], ib, 0)),
    ]
    args = [bexp, xd, Wg, Wu, Wd]
    if not first:
        in_specs.append(pl.BlockSpec((B, H), lambda b, be: (b, 0)))
        args.append(prev)
    grid_spec = pltpu.PrefetchScalarGridSpec(
        num_scalar_prefetch=1,
        grid=(NBLK,),
        in_specs=in_specs,
        out_specs=pl.BlockSpec((B, H), lambda b, be: (b, 0)),
    )
    return pl.pallas_call(
        functools.partial(_gmm_half_body, first),
        grid_spec=grid_spec,
        out_shape=jax.ShapeDtypeStruct((R_ROWS, H), jnp.float32),
        compiler_params=pltpu.CompilerParams(
            dimension_semantics=("arbitrary",),
            vmem_limit_bytes=_VMEM_LIMIT),
    )(*args)


def _gmm(bexp, xd, Wg, Wu, Wd):
    eo0 = _gmm_half(0, bexp, xd, Wg, Wu, Wd)
    return _gmm_half(1, bexp, xd, Wg, Wu, Wd, prev=eo0)


def _shared_body(x_ref, wg_ref, wu_ref, wd_ref, out_ref):
    bf = jnp.bfloat16
    xb = x_ref[...].astype(bf)
    g = jnp.dot(xb, wg_ref[...], preferred_element_type=jnp.float32)
    u = jnp.dot(xb, wu_ref[...], preferred_element_type=jnp.float32)
    h = (g * jax.nn.sigmoid(g) * u).astype(bf)
    out_ref[...] = jnp.dot(h, wd_ref[...], preferred_element_type=jnp.float32)


def _shared(x, Wg_s, Wu_s, Wd_s):
    blk = 256
    return pl.pallas_call(
        _shared_body,
        grid=(N // blk,),
        in_specs=[
            pl.BlockSpec((blk, H), lambda t: (t, 0)),
            pl.BlockSpec((H, I), lambda t: (0, 0)),
            pl.BlockSpec((H, I), lambda t: (0, 0)),
            pl.BlockSpec((I, H), lambda t: (0, 0)),
        ],
        out_specs=pl.BlockSpec((blk, H), lambda t: (t, 0)),
        out_shape=jax.ShapeDtypeStruct((N, H), jnp.float32),
        compiler_params=pltpu.CompilerParams(
            dimension_semantics=("arbitrary",),
            vmem_limit_bytes=_VMEM_LIMIT),
    )(x, Wg_s, Wu_s, Wd_s)


def _combine_body(sh_ref, g1_ref, g2_ref, s1_ref, s2_ref, out_ref):
    w1 = s1_ref[:, 0:1]
    w2 = s2_ref[:, 0:1]
    out_ref[...] = sh_ref[...] + w1 * g1_ref[...] + w2 * g2_ref[...]


def _combine(sh, G, S1, S2):
    blk = 512
    return pl.pallas_call(
        _combine_body,
        grid=(N // blk,),
        in_specs=[
            pl.BlockSpec((blk, H), lambda t: (t, 0)),
            pl.BlockSpec((blk, H), lambda t: (t, 0)),
            pl.BlockSpec((blk, H), lambda t: (t + N // blk, 0)),
            pl.BlockSpec((blk, EPAD), lambda t: (t, 0)),
            pl.BlockSpec((blk, EPAD), lambda t: (t, 0)),
        ],
        out_specs=pl.BlockSpec((blk, H), lambda t: (t, 0)),
        out_shape=jax.ShapeDtypeStruct((N, H), jnp.float32),
        compiler_params=pltpu.CompilerParams(
            dimension_semantics=("parallel",)),
    )(sh, G, G, S1, S2)


def kernel(x, Wg_s, Wu_s, Wd_s, Wg, Wu, Wd, Wr, rbias):
    Wr_pad = jnp.zeros((H, EPAD), jnp.float32).at[:, :E_ROUTED].set(Wr)
    rb_pad = jnp.full((1, EPAD), -1e30, jnp.float32).at[0, :E_ROUTED].set(rbias)

    bf = jnp.bfloat16
    E1, E2, S1, S2 = _router(x, Wr_pad, rb_pad)
    eids = jnp.concatenate([E1[:, 0], E2[:, 0]])
    rt, pos2d, bexp = _dispatch(eids)

    x3 = x.reshape(N, 8, 128)
    xd = _gather_rows(x3, rt, R_ROWS, 24).reshape(R_ROWS, H)
    eo = _gmm(bexp, xd, Wg, Wu, Wd)
    sh = _shared(x, Wg_s.astype(bf), Wu_s.astype(bf), Wd_s.astype(bf))
    G = _gather_rows(eo.reshape(R_ROWS, 8, 128), pos2d.reshape(-1),
                     ASG, 32).reshape(ASG, H)
    return _combine(sh, G, S1, S2)
